# baseline (speedup 1.0000x reference)
import functools

import jax
import jax.numpy as jnp
from jax import lax
from jax.experimental import pallas as pl
from jax.experimental.pallas import tpu as pltpu, tpu_sc as plsc

N = 50000
E = 65536
D = 128
NR = 9
NP = 50176
RNG = 6272
RPT = 392
NPASS = 4
EPT = 4096
CH = 16384
SRC_MAGIC = sum(st << (2 * r) for r, st in enumerate([0, 0, 0, 1, 1, 2, 2, 2, 1]))
DST_OF_REL = [0, 1, 2, 2, 0, 0, 1, 2, 1]
RELS_OF_DST = [[0, 4, 5], [1, 6, 8], [2, 3, 7]]

_mesh = plsc.VectorSubcoreMesh(core_axis_name="c", subcore_axis_name="s",
                               num_cores=2, num_subcores=16)
_sc_params = pltpu.CompilerParams(needs_layout_passes=False)


def _zero16(ref, n, off=0):
    z = jnp.zeros((16,), ref.dtype)

    def body(i, _):
        ref[pl.ds(off + i * 16, 16)] = z
        return 0

    lax.fori_loop(0, n // 16, body, 0)


def _newton_rsqrt(d):
    i = plsc.bitcast(d, jnp.int32)
    i = jnp.int32(0x5F3759DF) - (i >> 1)
    y = plsc.bitcast(i, jnp.float32)
    for _ in range(3):
        y = y * (1.5 - 0.5 * d * y * y)
    return y


@functools.partial(
    pl.kernel,
    out_type=[
        jax.ShapeDtypeStruct((NR, E), jnp.float32),
        jax.ShapeDtypeStruct((NR, NP), jnp.float32),
    ],
    mesh=_mesh,
    compiler_params=_sc_params,
    scratch_types=[
        pltpu.VMEM((NP,), jnp.float32),
        pltpu.VMEM((CH,), jnp.int32),
        pltpu.VMEM((CH,), jnp.float32),
        pltpu.VMEM((CH,), jnp.float32),
    ],
)
def _deg_coef_kernel(eidx, ew, coef_out, indeg_out, hist, idx_v, ew_v, coef_v):
    cid = lax.axis_index("c")
    sid = lax.axis_index("s")
    wid = sid * 2 + cid
    ones = jnp.ones((16,), jnp.float32)

    def histogram(rel, which):
        _zero16(hist, NP)

        def chunk_body(c, _):
            pltpu.sync_copy(eidx.at[rel, which, pl.ds(c * CH, CH)], idx_v)

            def vec_body(j, _):
                idx = idx_v[pl.ds(j * 16, 16)]
                plsc.addupdate_scatter(hist, [idx], ones)
                return 0

            lax.fori_loop(0, CH // 16, vec_body, 0)
            return 0

        lax.fori_loop(0, E // CH, chunk_body, 0)

    @pl.when(wid < NR)
    def _():
        rel = wid
        histogram(rel, 0)

        def rs_body(i, _):
            d = hist[pl.ds(i * 16, 16)]
            hist[pl.ds(i * 16, 16)] = _newton_rsqrt(jnp.maximum(d, 1.0))
            return 0

        lax.fori_loop(0, NP // 16, rs_body, 0)

        def chunk_body(c, _):
            pltpu.sync_copy(eidx.at[rel, 0, pl.ds(c * CH, CH)], idx_v)
            pltpu.sync_copy(ew.at[rel, pl.ds(c * CH, CH)], ew_v)

            def vec_body(j, _):
                idx = idx_v[pl.ds(j * 16, 16)]
                g = plsc.load_gather(hist, [idx])
                coef_v[pl.ds(j * 16, 16)] = g * ew_v[pl.ds(j * 16, 16)]
                return 0

            lax.fori_loop(0, CH // 16, vec_body, 0)
            pltpu.sync_copy(coef_v, coef_out.at[rel, pl.ds(c * CH, CH)])
            return 0

        lax.fori_loop(0, E // CH, chunk_body, 0)

    @pl.when((wid >= 16) & (wid < 16 + NR))
    def _():
        rel = wid - 16
        histogram(rel, 1)
        pltpu.sync_copy(hist, indeg_out.at[rel])


@functools.partial(
    pl.kernel,
    out_type=jax.ShapeDtypeStruct((NR, NP, D), jnp.float32),
    mesh=_mesh,
    compiler_params=_sc_params,
    scratch_types=[
        pltpu.VMEM((EPT,), jnp.int32),
        pltpu.VMEM((EPT,), jnp.int32),
        pltpu.VMEM((EPT,), jnp.float32),
        pltpu.VMEM((EPT + 16,), jnp.int32),
        pltpu.VMEM((EPT + 16,), jnp.int32),
        pltpu.VMEM((EPT + 16,), jnp.float32),
        pltpu.VMEM((128,), jnp.int32),
        pltpu.VMEM((128,), jnp.int32),
        pltpu.VMEM((128, D), jnp.float32),
        pltpu.VMEM((128, D), jnp.float32),
        pltpu.VMEM_SHARED((RNG, D), jnp.float32),
        pltpu.SemaphoreType.DMA,
    ],
)
def _msg_kernel(x3, eidx, coef, agg_out,
                src_v, dst_v, coef_v, csrc, cdst, ccoef,
                bsrc, bdst, rows, zblk, acc, sem):
    cid = lax.axis_index("c")
    sid = lax.axis_index("s")

    def zb_body(i, _):
        r = i // 8
        k = i % 8
        zblk[r, pl.ds(k * 16, 16)] = jnp.zeros((16,), jnp.float32)
        return 0

    lax.fori_loop(0, 128 * 8, zb_body, 0)

    def iter_body(it, _):
        pss = it // NR
        rel = it - pss * NR
        st = (SRC_MAGIC >> (2 * rel)) & 3
        src_base = st * N
        base = (cid * NPASS + pss) * RNG

        for k in range(3):
            pltpu.sync_copy(zblk, acc.at[pl.ds(sid * RPT + k * 128, 128)])
        pltpu.sync_copy(zblk.at[pl.ds(0, 8)],
                        acc.at[pl.ds(sid * RPT + 384, 8)])
        plsc.subcore_barrier()

        off = sid * EPT
        pltpu.sync_copy(eidx.at[rel, 0, pl.ds(off, EPT)], src_v)
        pltpu.sync_copy(eidx.at[rel, 1, pl.ds(off, EPT)], dst_v)
        pltpu.sync_copy(coef.at[rel, pl.ds(off, EPT)], coef_v)

        _zero16(csrc, EPT + 16)
        _zero16(cdst, EPT + 16)
        _zero16(ccoef, EPT + 16)

        def cmp_body(j, cnt):
            s = src_v[pl.ds(j * 16, 16)]
            d = dst_v[pl.ds(j * 16, 16)]
            cf = coef_v[pl.ds(j * 16, 16)]
            ld = d - base
            mask = (ld >= 0) & (ld < RNG)
            plsc.store_compressed(csrc.at[pl.ds(cnt, 16)], s + src_base,
                                  mask=mask)
            plsc.store_compressed(cdst.at[pl.ds(cnt, 16)], ld, mask=mask)
            plsc.store_compressed(ccoef.at[pl.ds(cnt, 16)], cf, mask=mask)
            return cnt + jnp.sum(mask.astype(jnp.int32))

        cnt = lax.fori_loop(0, EPT // 16, cmp_body, 0)

        def batch_body(b, _):
            @pl.when(b * 128 < cnt)
            def _():
                for k in range(8):
                    bsrc[pl.ds(k * 16, 16)] = csrc[pl.ds(b * 128 + k * 16, 16)]
                    bdst[pl.ds(k * 16, 16)] = cdst[pl.ds(b * 128 + k * 16, 16)]
                pltpu.async_copy(x3.at[bsrc], rows, sem).wait()

                def row_body(r, _):
                    cvec = plsc.load_gather(
                        ccoef, [jnp.full((16,), b * 128 + r, jnp.int32)])
                    for k in range(8):
                        rows[r, pl.ds(k * 16, 16)] = (
                            rows[r, pl.ds(k * 16, 16)] * cvec)
                    return 0

                lax.fori_loop(0, 128, row_body, 0)
                pltpu.sync_copy(rows, acc.at[bdst], add=True)

            return 0

        lax.fori_loop(0, EPT // 128, batch_body, 0)
        plsc.subcore_barrier()

        for k in range(3):
            pltpu.sync_copy(
                acc.at[pl.ds(sid * RPT + k * 128, 128)],
                agg_out.at[rel, pl.ds(base + sid * RPT + k * 128, 128)])
        pltpu.sync_copy(
            acc.at[pl.ds(sid * RPT + 384, 8)],
            agg_out.at[rel, pl.ds(base + sid * RPT + 384, 8)])
        plsc.subcore_barrier()
        return 0

    lax.fori_loop(0, NPASS * NR, iter_body, 0)


_BLK = 512


def _epilogue_body(agg_ref, indeg_ref, w_ref, b_ref, out_ref):
    indeg = indeg_ref[...]
    scale = lax.rsqrt(jnp.maximum(indeg, 1.0))
    for t in range(3):
        rels = RELS_OF_DST[t]
        a = jnp.concatenate(
            [agg_ref[r] * scale[r][:, None] for r in rels], axis=1)
        w = jnp.concatenate([w_ref[r] for r in rels], axis=0)
        bias = b_ref[rels[0]] + b_ref[rels[1]] + b_ref[rels[2]]
        out_ref[t] = jax.nn.relu(
            jnp.dot(a, w, preferred_element_type=jnp.float32) + bias[None, :])


def _epilogue(agg, indeg, W, b):
    return pl.pallas_call(
        _epilogue_body,
        grid=(NP // _BLK,),
        in_specs=[
            pl.BlockSpec((NR, _BLK, D), lambda i: (0, i, 0)),
            pl.BlockSpec((NR, _BLK), lambda i: (0, i)),
            pl.BlockSpec((NR, D, D), lambda i: (0, 0, 0)),
            pl.BlockSpec((NR, D), lambda i: (0, 0)),
        ],
        out_specs=pl.BlockSpec((3, _BLK, D), lambda i: (0, i, 0)),
        out_shape=jax.ShapeDtypeStruct((3, NP, D), jnp.float32),
    )(agg, indeg, W, b)


@jax.jit
def kernel(x_mention, x_entity, x_sentence, W, b,
           edge_index_M_M, edge_index_M_E, edge_index_M_S,
           edge_index_E_S, edge_index_E_M, edge_index_S_M,
           edge_index_S_E, edge_index_S_S, edge_index_E_E,
           edge_weight_M_M, edge_weight_M_E, edge_weight_M_S,
           edge_weight_E_S, edge_weight_E_M, edge_weight_S_M,
           edge_weight_S_E, edge_weight_S_S, edge_weight_E_E):
    eidx = jnp.stack([
        edge_index_M_M, edge_index_M_E, edge_index_M_S, edge_index_E_S,
        edge_index_E_M, edge_index_S_M, edge_index_S_E, edge_index_S_S,
        edge_index_E_E])
    ew = jnp.stack([
        edge_weight_M_M, edge_weight_M_E, edge_weight_M_S, edge_weight_E_S,
        edge_weight_E_M, edge_weight_S_M, edge_weight_S_E, edge_weight_S_S,
        edge_weight_E_E])
    x3 = jnp.concatenate([x_mention, x_entity, x_sentence], axis=0)

    coef, indeg = _deg_coef_kernel(eidx, ew)
    agg = _msg_kernel(x3, eidx, coef)
    out = _epilogue(agg, indeg, W, b)
    return out[:, :N, :]
